# Initial kernel scaffold; baseline (speedup 1.0000x reference)
import jax
import jax.numpy as jnp
from jax.experimental import pallas as pl


def kernel(inputs, edge_index, labels, W_self0, b_self0, W_neigh0, b_neigh0, W_self1, b_self1, W_neigh1, b_neigh1, W_dec, b_dec):
    raise NotImplementedError("write your pallas kernel here")


# speedup vs baseline: 5.6407x; 5.6407x over previous
import functools

import jax
import jax.numpy as jnp
from jax import lax
from jax.experimental import pallas as pl
from jax.experimental.pallas import tpu as pltpu
from jax.experimental.pallas import tpu_sc as plsc

NN = 10000
EE = 320000
DD = 128

NC = 2
NS = 16
NW = NC * NS
CHUNK = 128
NCH = 79
EPW = CHUNK * NCH
EPAD = EPW * NW
RPT = 632
NACC = RPT * NS


def _sc_agg_body(table, srcs, dsts, out, deg_out,
                 acc_sh, deg_sh, src_v, dst_v, rows_v, ones_v, zdeg_v, sem):
    c = lax.axis_index("c")
    s = lax.axis_index("s")
    wid = s * NC + c
    r0 = s * RPT

    def zrows(i, _):
        rows_v[i // 8, pl.ds((i % 8) * 16, 16)] = jnp.zeros((16,), jnp.float32)
        return 0
    lax.fori_loop(0, CHUNK * 8, zrows, 0)

    def zdeg(i, _):
        zdeg_v[i] = jnp.zeros((16,), jnp.float32)
        return 0
    lax.fori_loop(0, RPT, zdeg, 0)

    def fones(i, _):
        ones_v[i] = jnp.ones((16,), jnp.float32)
        return 0
    lax.fori_loop(0, CHUNK, fones, 0)

    for r in range(4):
        pltpu.sync_copy(rows_v, acc_sh.at[pl.ds(r0 + r * CHUNK, CHUNK)])
    pltpu.sync_copy(rows_v.at[pl.ds(0, RPT - 4 * CHUNK)],
                    acc_sh.at[pl.ds(r0 + 4 * CHUNK, RPT - 4 * CHUNK)])
    pltpu.sync_copy(zdeg_v, deg_sh.at[pl.ds(r0, RPT)])
    plsc.subcore_barrier()

    def chunk_body(i, _):
        base = pl.multiple_of(wid * EPW + i * CHUNK, CHUNK)
        pltpu.sync_copy(srcs.at[pl.ds(base, CHUNK)], src_v)
        pltpu.sync_copy(dsts.at[pl.ds(base, CHUNK)], dst_v)
        pltpu.async_copy(table.at[src_v], rows_v, sem).wait()
        pltpu.sync_copy(rows_v, acc_sh.at[dst_v], add=True)
        pltpu.sync_copy(ones_v, deg_sh.at[dst_v], add=True)
        return 0
    lax.fori_loop(0, NCH, chunk_body, 0)
    plsc.subcore_barrier()

    pltpu.sync_copy(acc_sh.at[pl.ds(r0, RPT)], out.at[c, pl.ds(r0, RPT)])
    pltpu.sync_copy(deg_sh.at[pl.ds(r0, RPT)], deg_out.at[c, pl.ds(r0, RPT)])


@functools.cache
def _sc_agg():
    return pl.kernel(
        _sc_agg_body,
        out_type=(
            jax.ShapeDtypeStruct((NC, NACC, DD), jnp.float32),
            jax.ShapeDtypeStruct((NC, NACC, 16), jnp.float32),
        ),
        mesh=plsc.VectorSubcoreMesh(core_axis_name="c", subcore_axis_name="s",
                                    num_cores=NC, num_subcores=NS),
        scratch_types=[
            pltpu.VMEM_SHARED((NACC, DD), jnp.float32),
            pltpu.VMEM_SHARED((NACC, 16), jnp.float32),
            pltpu.VMEM((CHUNK,), jnp.int32),
            pltpu.VMEM((CHUNK,), jnp.int32),
            pltpu.VMEM((CHUNK, DD), jnp.float32),
            pltpu.VMEM((CHUNK, 16), jnp.float32),
            pltpu.VMEM((RPT, 16), jnp.float32),
            pltpu.SemaphoreType.DMA,
        ],
        compiler_params=pltpu.CompilerParams(use_tc_tiling_on_sc=False),
    )


def _layer_body(h_ref, p0_ref, p1_ref, d0_ref, d1_ref,
                ws_ref, wn_ref, bs_ref, bn_ref, o_ref, *, relu):
    deg = jnp.maximum(d0_ref[:, 0:1] + d1_ref[:, 0:1], 1.0)
    agg = (p0_ref[...] + p1_ref[...]) / deg
    o = (jnp.dot(h_ref[...], ws_ref[...], preferred_element_type=jnp.float32)
         + jnp.dot(agg, wn_ref[...], preferred_element_type=jnp.float32)
         + bs_ref[...] + bn_ref[...])
    o_ref[...] = jnp.maximum(o, 0.0) if relu else o


def _sage_layer(h, p0, p1, d0, d1, ws, wn, bs, bn, relu):
    B = 1000
    grid = NN // B
    return pl.pallas_call(
        functools.partial(_layer_body, relu=relu),
        grid=(grid,),
        in_specs=[
            pl.BlockSpec((B, DD), lambda i: (i, 0)),
            pl.BlockSpec((B, DD), lambda i: (i, 0)),
            pl.BlockSpec((B, DD), lambda i: (i, 0)),
            pl.BlockSpec((B, 16), lambda i: (i, 0)),
            pl.BlockSpec((B, 16), lambda i: (i, 0)),
            pl.BlockSpec((DD, DD), lambda i: (0, 0)),
            pl.BlockSpec((DD, DD), lambda i: (0, 0)),
            pl.BlockSpec((1, DD), lambda i: (0, 0)),
            pl.BlockSpec((1, DD), lambda i: (0, 0)),
        ],
        out_specs=pl.BlockSpec((B, DD), lambda i: (i, 0)),
        out_shape=jax.ShapeDtypeStruct((NN, DD), jnp.float32),
    )(h, p0, p1, d0, d1, ws, wn, bs, bn)


def _dec_body(hb_ref, h_ref, wd_ref, bd_ref, o_ref):
    z = (jnp.dot(hb_ref[...], wd_ref[...], preferred_element_type=jnp.float32)
         + bd_ref[...])
    a = lax.dot_general(z, h_ref[...], (((1,), (1,)), ((), ())),
                        preferred_element_type=jnp.float32)
    a = jnp.maximum(a, 0.0)
    m = jnp.max(a, axis=1, keepdims=True)
    e = jnp.exp(a - m)
    o_ref[...] = e / jnp.sum(e, axis=1, keepdims=True)


def _decoder(h, wd, bd):
    B = 400
    grid = NN // B
    return pl.pallas_call(
        _dec_body,
        grid=(grid,),
        in_specs=[
            pl.BlockSpec((B, DD), lambda i: (i, 0)),
            pl.BlockSpec((NN, DD), lambda i: (0, 0)),
            pl.BlockSpec((DD, DD), lambda i: (0, 0)),
            pl.BlockSpec((1, DD), lambda i: (0, 0)),
        ],
        out_specs=pl.BlockSpec((B, NN), lambda i: (i, 0)),
        out_shape=jax.ShapeDtypeStruct((NN, NN), jnp.float32),
    )(h, h, wd, bd)


def kernel(inputs, edge_index, labels, W_self0, b_self0, W_neigh0, b_neigh0,
           W_self1, b_self1, W_neigh1, b_neigh1, W_dec, b_dec):
    npad = EPAD - EE
    src = jnp.concatenate([edge_index[0], jnp.zeros((npad,), jnp.int32)])
    dst = jnp.concatenate([edge_index[1], jnp.full((npad,), NN, jnp.int32)])

    parts0, degp = _sc_agg()(inputs, src, dst)
    d0 = degp[0, :NN]
    d1 = degp[1, :NN]
    h1 = _sage_layer(inputs, parts0[0, :NN], parts0[1, :NN], d0, d1,
                     W_self0, W_neigh0, b_self0.reshape(1, DD),
                     b_neigh0.reshape(1, DD), relu=True)

    parts1, _ = _sc_agg()(h1, src, dst)
    h2 = _sage_layer(h1, parts1[0, :NN], parts1[1, :NN], d0, d1,
                     W_self1, W_neigh1, b_self1.reshape(1, DD),
                     b_neigh1.reshape(1, DD), relu=False)

    adj = _decoder(h2, W_dec, b_dec.reshape(1, DD))
    return (adj, h2, labels)
